# baseline (speedup 1.0000x reference)
import jax
import jax.numpy as jnp
from jax import lax
from jax.experimental import pallas as pl
from jax.experimental.pallas import tpu as pltpu
from jax.experimental.pallas import tpu_sc as plsc

VOCAB = 400000
D = 50
DPW = 64
B = 4096
L = 350
NW = 32
RPT = B // NW
VPT = VOCAB // NW
CH = 500
INV_L = 1.0 / L

_SC_PARAMS = dict(
    mesh=plsc.VectorSubcoreMesh(core_axis_name="c", subcore_axis_name="s"),
    compiler_params=pltpu.CompilerParams(use_tc_tiling_on_sc=False,
                                         needs_layout_passes=False),
)


def _pack_sc_body(table_hbm, out_hbm, in_v, out_v):
    wid = lax.axis_index("s") * 2 + lax.axis_index("c")
    base = wid * VPT

    def chunk(c, carry):
        r0 = base + c * CH
        pltpu.sync_copy(table_hbm.at[pl.ds(r0, CH)], in_v)

        def row(r, carry2):
            a = in_v[r, pl.ds(0, 16)]
            b = in_v[r, pl.ds(16, 16)]
            cc = in_v[r, pl.ds(32, 16)]
            dd = in_v[r, pl.ds(34, 16)]
            out_v[r, pl.ds(0, 32)] = plsc.pack(
                a, b, format=plsc.PackFormat.INTERLEAVED)
            out_v[r, pl.ds(32, 32)] = plsc.pack(
                cc, dd, format=plsc.PackFormat.INTERLEAVED)
            return carry2

        lax.fori_loop(0, CH, row, 0)
        pltpu.sync_copy(out_v, out_hbm.at[pl.ds(r0, CH)])
        return carry

    lax.fori_loop(0, VPT // CH, chunk, 0)


_pack_sc = pl.kernel(
    _pack_sc_body,
    out_type=jax.ShapeDtypeStruct((VOCAB, DPW), jnp.bfloat16),
    scratch_types=[
        pltpu.VMEM((CH, D), jnp.float32),
        pltpu.VMEM((CH, DPW), jnp.bfloat16),
    ],
    **_SC_PARAMS,
)


def _glove_sc_body(table_hbm, idx_hbm, out_hbm, idx_v, rows0_v, rows1_v,
                   rows2_v, stage_v, sem0, sem1, sem2):
    wid = lax.axis_index("s") * 2 + lax.axis_index("c")
    base = wid * RPT

    pltpu.sync_copy(idx_hbm.at[pl.ds(base, RPT)], idx_v)

    def issue(b, buf, sem):
        pltpu.async_copy(table_hbm.at[idx_v.at[b, pl.ds(0, 128)]],
                         buf.at[pl.ds(0, 128)], sem)
        pltpu.async_copy(table_hbm.at[idx_v.at[b, pl.ds(128, 128)]],
                         buf.at[pl.ds(128, 128)], sem)
        pltpu.async_copy(table_hbm.at[idx_v.at[b, pl.ds(256, 94)]],
                         buf.at[pl.ds(256, 94)], sem)

    def issue_if(b, buf, sem):
        @pl.when(b < RPT)
        def _():
            issue(b, buf, sem)

    def drain(buf, sem):
        pltpu.make_async_copy(table_hbm.at[pl.ds(0, L)], buf, sem).wait()

    def acc_token(buf, t, a0, a1, a2, a3):
        u0, u1 = plsc.unpack(buf[t, pl.ds(0, 32)],
                             format=plsc.PackFormat.INTERLEAVED)
        u2, u3 = plsc.unpack(buf[t, pl.ds(32, 32)],
                             format=plsc.PackFormat.INTERLEAVED)
        return a0 + u0, a1 + u1, a2 + u2, a3 + u3

    def reduce_into(buf, b):
        def tok(t, acc):
            x0, x1, x2, x3, y0, y1, y2, y3 = acc
            t0 = t * 5
            x0, x1, x2, x3 = acc_token(buf, t0, x0, x1, x2, x3)
            y0, y1, y2, y3 = acc_token(buf, t0 + 1, y0, y1, y2, y3)
            x0, x1, x2, x3 = acc_token(buf, t0 + 2, x0, x1, x2, x3)
            y0, y1, y2, y3 = acc_token(buf, t0 + 3, y0, y1, y2, y3)
            x0, x1, x2, x3 = acc_token(buf, t0 + 4, x0, x1, x2, x3)
            return (x0, x1, x2, x3, y0, y1, y2, y3)

        z = jnp.zeros((16,), jnp.float32)
        x0, x1, x2, x3, y0, y1, y2, y3 = lax.fori_loop(
            0, L // 5, tok, (z,) * 8)
        s = jnp.float32(INV_L)
        stage_v[b, pl.ds(0, 16)] = (x0 + y0) * s
        stage_v[b, pl.ds(16, 16)] = (x1 + y1) * s
        stage_v[b, pl.ds(32, 16)] = (x2 + y2) * s
        stage_v[b, pl.ds(34, 16)] = (x3 + y3) * s

    issue(0, rows0_v, sem0)
    issue(1, rows1_v, sem1)
    issue(2, rows2_v, sem2)

    def tri_body(i, carry):
        r = 3 * i
        drain(rows0_v, sem0)
        reduce_into(rows0_v, r)
        issue_if(r + 3, rows0_v, sem0)
        drain(rows1_v, sem1)
        reduce_into(rows1_v, r + 1)
        issue_if(r + 4, rows1_v, sem1)
        drain(rows2_v, sem2)
        reduce_into(rows2_v, r + 2)
        issue_if(r + 5, rows2_v, sem2)
        return carry

    lax.fori_loop(0, RPT // 3, tri_body, 0)
    drain(rows0_v, sem0)
    reduce_into(rows0_v, RPT - 2)
    drain(rows1_v, sem1)
    reduce_into(rows1_v, RPT - 1)
    pltpu.sync_copy(stage_v, out_hbm.at[pl.ds(base, RPT)])


_glove_sc = pl.kernel(
    _glove_sc_body,
    out_type=jax.ShapeDtypeStruct((B, D), jnp.float32),
    scratch_types=[
        pltpu.VMEM((RPT, L), jnp.int32),
        pltpu.VMEM((L, DPW), jnp.bfloat16),
        pltpu.VMEM((L, DPW), jnp.bfloat16),
        pltpu.VMEM((L, DPW), jnp.bfloat16),
        pltpu.VMEM((RPT, D), jnp.float32),
        pltpu.SemaphoreType.DMA,
        pltpu.SemaphoreType.DMA,
        pltpu.SemaphoreType.DMA,
    ],
    **_SC_PARAMS,
)


def kernel(table, indices):
    table_bp = _pack_sc(table)
    idx32 = indices.astype(jnp.int32)
    return _glove_sc(table_bp, idx32)


# speedup vs baseline: 1.3417x; 1.3417x over previous
import jax
import jax.numpy as jnp
from jax import lax
from jax.experimental import pallas as pl
from jax.experimental.pallas import tpu as pltpu
from jax.experimental.pallas import tpu_sc as plsc

VOCAB = 400000
D = 50
DPW = 64
B = 4096
L = 350
LP = 352
NW = 32
RPT = B // NW
INV_L = 1.0 / L
HI16 = jnp.int32(-65536)


def _glove_sc_body(table_hbm, idx_hbm, out_hbm, idx_v, rows0_v, rows1_v,
                   rows2_v, stage_v, sem0, sem1, sem2):
    wid = lax.axis_index("s") * 2 + lax.axis_index("c")
    base = wid * RPT

    pltpu.sync_copy(idx_hbm.at[pl.ds(base, RPT)], idx_v)

    lanes = lax.iota(jnp.int32, 16)
    col_e0 = 2 * lanes
    col_o0 = 2 * lanes + 1
    col_e1 = 32 + 2 * lanes
    col_o1 = 33 + 2 * lanes
    tmask = lanes < 9

    def issue(b, buf, sem):
        pltpu.async_copy(table_hbm.at[idx_v.at[b, pl.ds(0, 128)]],
                         buf.at[pl.ds(0, 128)], sem)
        pltpu.async_copy(table_hbm.at[idx_v.at[b, pl.ds(128, 128)]],
                         buf.at[pl.ds(128, 128)], sem)
        pltpu.async_copy(table_hbm.at[idx_v.at[b, pl.ds(256, 94)]],
                         buf.at[pl.ds(256, 94)], sem)

    def issue_if(b, buf, sem):
        @pl.when(b < RPT)
        def _():
            issue(b, buf, sem)

    def drain(buf, sem):
        pltpu.make_async_copy(table_hbm.at[pl.ds(0, L)], buf, sem).wait()

    def acc_token(buf, t, ae, ao, be, bo):
        e0, o0 = plsc.unpack(buf[t, pl.ds(0, 32)],
                             format=plsc.PackFormat.INTERLEAVED)
        e1, o1 = plsc.unpack(buf[t, pl.ds(32, 32)],
                             format=plsc.PackFormat.INTERLEAVED)
        return ae + e0, ao + o0, be + e1, bo + o1

    def reduce_into(buf, b):
        def tok(t, acc):
            ae0, ao0, be0, bo0, ae1, ao1, be1, bo1 = acc
            t0 = t * 5
            ae0, ao0, be0, bo0 = acc_token(buf, t0, ae0, ao0, be0, bo0)
            ae1, ao1, be1, bo1 = acc_token(buf, t0 + 1, ae1, ao1, be1, bo1)
            ae0, ao0, be0, bo0 = acc_token(buf, t0 + 2, ae0, ao0, be0, bo0)
            ae1, ao1, be1, bo1 = acc_token(buf, t0 + 3, ae1, ao1, be1, bo1)
            ae0, ao0, be0, bo0 = acc_token(buf, t0 + 4, ae0, ao0, be0, bo0)
            return (ae0, ao0, be0, bo0, ae1, ao1, be1, bo1)

        z = jnp.zeros((16,), jnp.float32)
        ae0, ao0, be0, bo0, ae1, ao1, be1, bo1 = lax.fori_loop(
            0, L // 5, tok, (z,) * 8)
        s = jnp.float32(INV_L)
        row = jnp.full((16,), b, jnp.int32)
        plsc.store_scatter(stage_v, [row, col_e0], (ae0 + ae1) * s)
        plsc.store_scatter(stage_v, [row, col_o0], (ao0 + ao1) * s)
        plsc.store_scatter(stage_v, [row, col_e1], (be0 + be1) * s, mask=tmask)
        plsc.store_scatter(stage_v, [row, col_o1], (bo0 + bo1) * s, mask=tmask)

    issue(0, rows0_v, sem0)
    issue(1, rows1_v, sem1)
    issue(2, rows2_v, sem2)

    def tri_body(i, carry):
        r = 3 * i
        drain(rows0_v, sem0)
        reduce_into(rows0_v, r)
        issue_if(r + 3, rows0_v, sem0)
        drain(rows1_v, sem1)
        reduce_into(rows1_v, r + 1)
        issue_if(r + 4, rows1_v, sem1)
        drain(rows2_v, sem2)
        reduce_into(rows2_v, r + 2)
        issue_if(r + 5, rows2_v, sem2)
        return carry

    lax.fori_loop(0, RPT // 3, tri_body, 0)
    drain(rows0_v, sem0)
    reduce_into(rows0_v, RPT - 2)
    drain(rows1_v, sem1)
    reduce_into(rows1_v, RPT - 1)
    pltpu.sync_copy(stage_v, out_hbm.at[pl.ds(base, RPT)])


_glove_sc = pl.kernel(
    _glove_sc_body,
    out_type=jax.ShapeDtypeStruct((B, D), jnp.float32),
    mesh=plsc.VectorSubcoreMesh(core_axis_name="c", subcore_axis_name="s"),
    compiler_params=pltpu.CompilerParams(use_tc_tiling_on_sc=False,
                                         needs_layout_passes=False),
    scratch_types=[
        pltpu.VMEM((RPT, L), jnp.int32),
        pltpu.VMEM((L, DPW), jnp.bfloat16),
        pltpu.VMEM((L, DPW), jnp.bfloat16),
        pltpu.VMEM((L, DPW), jnp.bfloat16),
        pltpu.VMEM((RPT, D), jnp.float32),
        pltpu.SemaphoreType.DMA,
        pltpu.SemaphoreType.DMA,
        pltpu.SemaphoreType.DMA,
    ],
)


def kernel(table, indices):
    table_bp = jnp.pad(table.astype(jnp.bfloat16), ((0, 0), (0, DPW - D)))
    idx32 = indices.astype(jnp.int32)
    return _glove_sc(table_bp, idx32)
